# baseline (speedup 1.0000x reference)
import functools

import jax
import jax.numpy as jnp
from jax import lax
from jax.experimental import pallas as pl
from jax.experimental.pallas import tpu as pltpu
from jax.experimental.pallas import tpu_sc as plsc

N = 10000
D = 128
G = 64

NC = 2
NS = 16
NT = NC * NS
CH = 104

ACC_ROWS = 10112
RPT = ACC_ROWS // NS


def _sc_agg(h, srcg, dstg, zeros, ept_ch):
    mesh = plsc.VectorSubcoreMesh(core_axis_name="c", subcore_axis_name="s")

    @functools.partial(
        pl.kernel,
        mesh=mesh,
        out_type=jax.ShapeDtypeStruct((NC, ACC_ROWS, D), jnp.float32),
        scratch_types=[
            pltpu.VMEM((ept_ch * CH,), jnp.int32),
            pltpu.VMEM((ept_ch, CH), jnp.int32),
            pltpu.VMEM((2, CH, D), jnp.float32),
            pltpu.VMEM_SHARED((ACC_ROWS, D), jnp.float32),
            pltpu.SemaphoreType.DMA,
            pltpu.SemaphoreType.DMA,
        ],
    )
    def agg(h_hbm, src_hbm, dst_hbm, z_hbm, out_hbm, sidx, didx, rows, acc,
            gsem0, gsem1):
        c = lax.axis_index("c")
        s = lax.axis_index("s")
        wid = c * NS + s
        pltpu.sync_copy(z_hbm.at[pl.ds(s * RPT, RPT)], acc.at[pl.ds(s * RPT, RPT)])
        pltpu.sync_copy(src_hbm.at[wid], sidx)
        pltpu.sync_copy(dst_hbm.at[wid], didx)
        plsc.subcore_barrier()

        def group(jg, carry):
            j0 = 2 * jg
            g0 = pltpu.async_copy(
                h_hbm.at[sidx.at[pl.ds(j0 * CH, CH)]], rows.at[0], gsem0)
            g1 = pltpu.async_copy(
                h_hbm.at[sidx.at[pl.ds((j0 + 1) * CH, CH)]], rows.at[1], gsem1)
            g0.wait()
            pltpu.sync_copy(rows.at[0], acc.at[didx.at[j0]], add=True)
            g1.wait()
            pltpu.sync_copy(rows.at[1], acc.at[didx.at[j0 + 1]], add=True)
            return carry

        lax.fori_loop(0, ept_ch // 2, group, 0)
        plsc.subcore_barrier()
        pltpu.sync_copy(acc.at[pl.ds(s * RPT, RPT)],
                        out_hbm.at[c, pl.ds(s * RPT, RPT)])

    return agg(h, srcg, dstg, zeros)


def _tc_pre(x, w, b2):
    def body(x_ref, w_ref, b_ref, o_ref):
        o_ref[...] = jnp.dot(x_ref[...], w_ref[...],
                             preferred_element_type=jnp.float32) + b_ref[...]

    return pl.pallas_call(
        body, out_shape=jax.ShapeDtypeStruct((N, D), jnp.float32)
    )(x, w, b2)


def _tc_layer(h, parts, w1, w2, g2, b2):
    def body(h_ref, p_ref, w1_ref, w2_ref, g_ref, b_ref, o_ref):
        t = h_ref[...] + p_ref[0, :N] + p_ref[1, :N]
        u = jnp.dot(t, w1_ref[...], preferred_element_type=jnp.float32)
        u = jnp.where(u >= 0, u, 0.01 * u)
        z = jnp.dot(u, w2_ref[...], preferred_element_type=jnp.float32)
        m = jnp.mean(z, axis=0, keepdims=True)
        cz = z - m
        v = jnp.mean(cz * cz, axis=0, keepdims=True)
        o_ref[...] = cz * lax.rsqrt(v + 1e-4) * g_ref[...] + b_ref[...]

    return pl.pallas_call(
        body, out_shape=jax.ShapeDtypeStruct((N, D), jnp.float32)
    )(h, parts, w1, w2, g2, b2)


def _tc_final(z0, z1, z2, bt, ff1, ff2, ff3, ffsc):
    def body(z0_ref, z1_ref, z2_ref, bt_ref, f1_ref, f2_ref, f3_ref,
             fsc_ref, o_ref):
        gids = lax.broadcasted_iota(jnp.int32, (G, N), 0)
        oh = (bt_ref[...] == gids).astype(jnp.float32)
        y0 = jnp.dot(oh, z0_ref[...], preferred_element_type=jnp.float32)
        y1 = jnp.dot(oh, z1_ref[...], preferred_element_type=jnp.float32)
        y2 = jnp.dot(oh, z2_ref[...], preferred_element_type=jnp.float32)
        y = jnp.concatenate([y0, y1, y2], axis=1)

        def lk(v):
            return jnp.where(v >= 0, v, 0.01 * v)

        blk = lk(jnp.dot(y, f1_ref[...], preferred_element_type=jnp.float32))
        blk = lk(jnp.dot(blk, f2_ref[...], preferred_element_type=jnp.float32))
        blk = lk(jnp.dot(blk, f3_ref[...], preferred_element_type=jnp.float32))
        o_ref[...] = blk + jnp.dot(y, fsc_ref[...],
                                   preferred_element_type=jnp.float32)

    return pl.pallas_call(
        body, out_shape=jax.ShapeDtypeStruct((G, 3 * D), jnp.float32)
    )(z0, z1, z2, bt, ff1, ff2, ff3, ffsc)


def kernel(x, edge_index, batch, pre_W, pre_b, w1_0, w2_0, g_0, b_0,
           w1_1, w2_1, g_1, b_1, w1_2, w2_2, g_2, b_2, ff1, ff2, ff3, ffsc):
    e = edge_index.shape[1]
    ept_ch = -(-e // (NT * CH * 2)) * 2
    epad = ept_ch * CH * NT
    src = edge_index[0].astype(jnp.int32)
    dst = edge_index[1].astype(jnp.int32)
    srcg = jnp.pad(src, (0, epad - e)).reshape(NT, ept_ch * CH)
    dstg = jnp.pad(dst, (0, epad - e), constant_values=N).reshape(NT, ept_ch, CH)
    zeros = jnp.zeros((ACC_ROWS, D), jnp.float32)

    h = _tc_pre(x, pre_W, pre_b.reshape(1, D))
    layers = [(w1_0, w2_0, g_0, b_0), (w1_1, w2_1, g_1, b_1),
              (w1_2, w2_2, g_2, b_2)]
    zs = []
    for (w1, w2, g, b) in layers:
        parts = _sc_agg(h, srcg, dstg, zeros, ept_ch)
        h = _tc_layer(h, parts, w1, w2, g.reshape(1, D), b.reshape(1, D))
        zs.append(h)

    return _tc_final(zs[0], zs[1], zs[2],
                     batch.reshape(1, N).astype(jnp.int32),
                     ff1, ff2, ff3, ffsc)


# speedup vs baseline: 1.0107x; 1.0107x over previous
import functools

import jax
import jax.numpy as jnp
from jax import lax
from jax.experimental import pallas as pl
from jax.experimental.pallas import tpu as pltpu
from jax.experimental.pallas import tpu_sc as plsc

N = 10000
D = 128
G = 64

NC = 2
NS = 16
NT = NC * NS
CH = 104

ACC_ROWS = 10112
RPT = ACC_ROWS // NS


def _sc_agg(h, srcg, dstg, zeros, ept_ch):
    mesh = plsc.VectorSubcoreMesh(core_axis_name="c", subcore_axis_name="s")

    @functools.partial(
        pl.kernel,
        mesh=mesh,
        out_type=jax.ShapeDtypeStruct((NC, ACC_ROWS, D), jnp.float32),
        scratch_types=[
            pltpu.VMEM((ept_ch * CH,), jnp.int32),
            pltpu.VMEM((ept_ch, CH), jnp.int32),
            pltpu.VMEM((2, CH, D), jnp.float32),
            pltpu.VMEM_SHARED((ACC_ROWS, D), jnp.float32),
            pltpu.SemaphoreType.DMA,
            pltpu.SemaphoreType.DMA,
        ],
    )
    def agg(h_hbm, src_hbm, dst_hbm, z_hbm, out_hbm, sidx, didx, rows, acc,
            gsem0, gsem1):
        c = lax.axis_index("c")
        s = lax.axis_index("s")
        wid = c * NS + s
        pltpu.sync_copy(z_hbm.at[pl.ds(s * RPT, RPT)], acc.at[pl.ds(s * RPT, RPT)])
        pltpu.sync_copy(src_hbm.at[wid], sidx)
        pltpu.sync_copy(dst_hbm.at[wid], didx)
        plsc.subcore_barrier()

        def group(jg, carry):
            j0 = 2 * jg
            g0 = pltpu.async_copy(
                h_hbm.at[sidx.at[pl.ds(j0 * CH, CH)]], rows.at[0], gsem0)
            g0.wait()
            g1 = pltpu.async_copy(
                h_hbm.at[sidx.at[pl.ds((j0 + 1) * CH, CH)]], rows.at[1], gsem1)
            pltpu.sync_copy(rows.at[0], acc.at[didx.at[j0]], add=True)
            g1.wait()
            pltpu.sync_copy(rows.at[1], acc.at[didx.at[j0 + 1]], add=True)
            return carry

        lax.fori_loop(0, ept_ch // 2, group, 0)
        plsc.subcore_barrier()
        pltpu.sync_copy(acc.at[pl.ds(s * RPT, RPT)],
                        out_hbm.at[c, pl.ds(s * RPT, RPT)])

    return agg(h, srcg, dstg, zeros)


def _tc_pre(x, w, b2):
    def body(x_ref, w_ref, b_ref, o_ref):
        o_ref[...] = jnp.dot(x_ref[...], w_ref[...],
                             preferred_element_type=jnp.float32) + b_ref[...]

    return pl.pallas_call(
        body, out_shape=jax.ShapeDtypeStruct((N, D), jnp.float32)
    )(x, w, b2)


def _tc_layer(h, parts, w1, w2, g2, b2):
    def body(h_ref, p_ref, w1_ref, w2_ref, g_ref, b_ref, o_ref):
        t = h_ref[...] + p_ref[0, :N] + p_ref[1, :N]
        u = jnp.dot(t, w1_ref[...], preferred_element_type=jnp.float32)
        u = jnp.where(u >= 0, u, 0.01 * u)
        z = jnp.dot(u, w2_ref[...], preferred_element_type=jnp.float32)
        m = jnp.mean(z, axis=0, keepdims=True)
        cz = z - m
        v = jnp.mean(cz * cz, axis=0, keepdims=True)
        o_ref[...] = cz * lax.rsqrt(v + 1e-4) * g_ref[...] + b_ref[...]

    return pl.pallas_call(
        body, out_shape=jax.ShapeDtypeStruct((N, D), jnp.float32)
    )(h, parts, w1, w2, g2, b2)


def _tc_final(z0, z1, z2, bt, ff1, ff2, ff3, ffsc):
    def body(z0_ref, z1_ref, z2_ref, bt_ref, f1_ref, f2_ref, f3_ref,
             fsc_ref, o_ref):
        gids = lax.broadcasted_iota(jnp.int32, (G, N), 0)
        oh = (bt_ref[...] == gids).astype(jnp.float32)
        y0 = jnp.dot(oh, z0_ref[...], preferred_element_type=jnp.float32)
        y1 = jnp.dot(oh, z1_ref[...], preferred_element_type=jnp.float32)
        y2 = jnp.dot(oh, z2_ref[...], preferred_element_type=jnp.float32)
        y = jnp.concatenate([y0, y1, y2], axis=1)

        def lk(v):
            return jnp.where(v >= 0, v, 0.01 * v)

        blk = lk(jnp.dot(y, f1_ref[...], preferred_element_type=jnp.float32))
        blk = lk(jnp.dot(blk, f2_ref[...], preferred_element_type=jnp.float32))
        blk = lk(jnp.dot(blk, f3_ref[...], preferred_element_type=jnp.float32))
        o_ref[...] = blk + jnp.dot(y, fsc_ref[...],
                                   preferred_element_type=jnp.float32)

    return pl.pallas_call(
        body, out_shape=jax.ShapeDtypeStruct((G, 3 * D), jnp.float32)
    )(z0, z1, z2, bt, ff1, ff2, ff3, ffsc)


def kernel(x, edge_index, batch, pre_W, pre_b, w1_0, w2_0, g_0, b_0,
           w1_1, w2_1, g_1, b_1, w1_2, w2_2, g_2, b_2, ff1, ff2, ff3, ffsc):
    e = edge_index.shape[1]
    ept_ch = -(-e // (NT * CH * 2)) * 2
    epad = ept_ch * CH * NT
    src = edge_index[0].astype(jnp.int32)
    dst = edge_index[1].astype(jnp.int32)
    srcg = jnp.pad(src, (0, epad - e)).reshape(NT, ept_ch * CH)
    dstg = jnp.pad(dst, (0, epad - e), constant_values=N).reshape(NT, ept_ch, CH)
    zeros = jnp.zeros((ACC_ROWS, D), jnp.float32)

    h = _tc_pre(x, pre_W, pre_b.reshape(1, D))
    layers = [(w1_0, w2_0, g_0, b_0), (w1_1, w2_1, g_1, b_1),
              (w1_2, w2_2, g_2, b_2)]
    zs = []
    for (w1, w2, g, b) in layers:
        parts = _sc_agg(h, srcg, dstg, zeros, ept_ch)
        h = _tc_layer(h, parts, w1, w2, g.reshape(1, D), b.reshape(1, D))
        zs.append(h)

    return _tc_final(zs[0], zs[1], zs[2],
                     batch.reshape(1, N).astype(jnp.int32),
                     ff1, ff2, ff3, ffsc)


# speedup vs baseline: 1.5975x; 1.5806x over previous
import functools

import jax
import jax.numpy as jnp
from jax import lax
from jax.experimental import pallas as pl
from jax.experimental.pallas import tpu as pltpu
from jax.experimental.pallas import tpu_sc as plsc

N = 10000
D = 128
G = 64

NC = 2
NS = 16
NT = NC * NS
CH = 128
SKEW0 = 0.375

ACC_ROWS = 10112
RPT = ACC_ROWS // NS


def _sc_agg(h, srcg, dstg, zeros, n0, n1):
    maxn = max(n0, n1)
    mesh = plsc.VectorSubcoreMesh(core_axis_name="c", subcore_axis_name="s")

    @functools.partial(
        pl.kernel,
        mesh=mesh,
        out_type=jax.ShapeDtypeStruct((NC, ACC_ROWS, D), jnp.float32),
        scratch_types=[
            pltpu.VMEM((maxn, CH), jnp.int32),
            pltpu.VMEM((maxn, CH), jnp.int32),
            pltpu.VMEM((CH, D), jnp.float32),
            pltpu.VMEM_SHARED((ACC_ROWS, D), jnp.float32),
            pltpu.SemaphoreType.DMA,
        ],
    )
    def agg(h_hbm, src_hbm, dst_hbm, z_hbm, out_hbm, sidx, didx, rows, acc,
            gsem):
        c = lax.axis_index("c")
        s = lax.axis_index("s")
        wid = c * NS + s
        pltpu.sync_copy(z_hbm.at[pl.ds(s * RPT, RPT)], acc.at[pl.ds(s * RPT, RPT)])
        pltpu.sync_copy(src_hbm.at[wid], sidx)
        pltpu.sync_copy(dst_hbm.at[wid], didx)
        plsc.subcore_barrier()

        def body(j, carry):
            pltpu.async_copy(h_hbm.at[sidx.at[j]], rows, gsem).wait()
            pltpu.sync_copy(rows, acc.at[didx.at[j]], add=True)
            return carry

        nch = lax.select(c == 0, n0, n1)
        lax.fori_loop(0, nch, body, 0)
        plsc.subcore_barrier()
        pltpu.sync_copy(acc.at[pl.ds(s * RPT, RPT)],
                        out_hbm.at[c, pl.ds(s * RPT, RPT)])

    return agg(h, srcg, dstg, zeros)


def _tc_pre(x, w, b2):
    def body(x_ref, w_ref, b_ref, o_ref):
        o_ref[...] = jnp.dot(x_ref[...], w_ref[...],
                             preferred_element_type=jnp.float32) + b_ref[...]

    return pl.pallas_call(
        body, out_shape=jax.ShapeDtypeStruct((N, D), jnp.float32)
    )(x, w, b2)


def _tc_layer(h, parts, w1, w2, g2, b2):
    def body(h_ref, p_ref, w1_ref, w2_ref, g_ref, b_ref, o_ref):
        t = h_ref[...] + p_ref[0, :N] + p_ref[1, :N]
        u = jnp.dot(t, w1_ref[...], preferred_element_type=jnp.float32)
        u = jnp.where(u >= 0, u, 0.01 * u)
        z = jnp.dot(u, w2_ref[...], preferred_element_type=jnp.float32)
        m = jnp.mean(z, axis=0, keepdims=True)
        cz = z - m
        v = jnp.mean(cz * cz, axis=0, keepdims=True)
        o_ref[...] = cz * lax.rsqrt(v + 1e-4) * g_ref[...] + b_ref[...]

    return pl.pallas_call(
        body, out_shape=jax.ShapeDtypeStruct((N, D), jnp.float32)
    )(h, parts, w1, w2, g2, b2)


def _tc_final(z0, z1, z2, bt, ff1, ff2, ff3, ffsc):
    def body(z0_ref, z1_ref, z2_ref, bt_ref, f1_ref, f2_ref, f3_ref,
             fsc_ref, o_ref):
        gids = lax.broadcasted_iota(jnp.int32, (G, N), 0)
        oh = (bt_ref[...] == gids).astype(jnp.float32)
        y0 = jnp.dot(oh, z0_ref[...], preferred_element_type=jnp.float32)
        y1 = jnp.dot(oh, z1_ref[...], preferred_element_type=jnp.float32)
        y2 = jnp.dot(oh, z2_ref[...], preferred_element_type=jnp.float32)
        y = jnp.concatenate([y0, y1, y2], axis=1)

        def lk(v):
            return jnp.where(v >= 0, v, 0.01 * v)

        blk = lk(jnp.dot(y, f1_ref[...], preferred_element_type=jnp.float32))
        blk = lk(jnp.dot(blk, f2_ref[...], preferred_element_type=jnp.float32))
        blk = lk(jnp.dot(blk, f3_ref[...], preferred_element_type=jnp.float32))
        o_ref[...] = blk + jnp.dot(y, fsc_ref[...],
                                   preferred_element_type=jnp.float32)

    return pl.pallas_call(
        body, out_shape=jax.ShapeDtypeStruct((G, 3 * D), jnp.float32)
    )(z0, z1, z2, bt, ff1, ff2, ff3, ffsc)


def kernel(x, edge_index, batch, pre_W, pre_b, w1_0, w2_0, g_0, b_0,
           w1_1, w2_1, g_1, b_1, w1_2, w2_2, g_2, b_2, ff1, ff2, ff3, ffsc):
    e = edge_index.shape[1]
    totch = -(-e // (NS * CH))
    n0 = max(2, round(totch * SKEW0))
    n1 = totch - n0
    maxn = max(n0, n1)

    def slab(a, fill):
        a = a.astype(jnp.int32)
        e0 = n0 * CH * NS
        b0 = a[:e0].reshape(NS, n0, CH)
        b0 = jnp.pad(b0, ((0, 0), (0, maxn - n0), (0, 0)),
                     constant_values=fill)
        b1 = jnp.pad(a[e0:], (0, n1 * CH * NS - (e - e0)),
                     constant_values=fill).reshape(NS, n1, CH)
        b1 = jnp.pad(b1, ((0, 0), (0, maxn - n1), (0, 0)),
                     constant_values=fill)
        return jnp.concatenate([b0, b1], axis=0)

    srcg = slab(edge_index[0], 0)
    dstg = slab(edge_index[1], N)
    zeros = jnp.zeros((ACC_ROWS, D), jnp.float32)

    h = _tc_pre(x, pre_W, pre_b.reshape(1, D))
    layers = [(w1_0, w2_0, g_0, b_0), (w1_1, w2_1, g_1, b_1),
              (w1_2, w2_2, g_2, b_2)]
    zs = []
    for (w1, w2, g, b) in layers:
        parts = _sc_agg(h, srcg, dstg, zeros, n0, n1)
        h = _tc_layer(h, parts, w1, w2, g.reshape(1, D), b.reshape(1, D))
        zs.append(h)

    return _tc_final(zs[0], zs[1], zs[2],
                     batch.reshape(1, N).astype(jnp.int32),
                     ff1, ff2, ff3, ffsc)


# speedup vs baseline: 1.8947x; 1.1860x over previous
import functools

import jax
import jax.numpy as jnp
from jax import lax
from jax.experimental import pallas as pl
from jax.experimental.pallas import tpu as pltpu
from jax.experimental.pallas import tpu_sc as plsc

N = 10000
D = 128
G = 64

NC = 2
NS = 16
NT = NC * NS
CH = 128
SKEW0 = 0.625

ACC_ROWS = 10112
RPT = ACC_ROWS // NS


def _sc_agg(h, srcg, dstg, zeros, n0, n1):
    maxn = max(n0, n1)
    mesh = plsc.VectorSubcoreMesh(core_axis_name="c", subcore_axis_name="s")

    @functools.partial(
        pl.kernel,
        mesh=mesh,
        out_type=jax.ShapeDtypeStruct((NC, ACC_ROWS, D), jnp.float32),
        scratch_types=[
            pltpu.VMEM((maxn, CH), jnp.int32),
            pltpu.VMEM((maxn, CH), jnp.int32),
            pltpu.VMEM((CH, D), jnp.float32),
            pltpu.VMEM_SHARED((ACC_ROWS, D), jnp.float32),
            pltpu.SemaphoreType.DMA,
        ],
    )
    def agg(h_hbm, src_hbm, dst_hbm, z_hbm, out_hbm, sidx, didx, rows, acc,
            gsem):
        c = lax.axis_index("c")
        s = lax.axis_index("s")
        wid = c * NS + s
        pltpu.sync_copy(z_hbm.at[pl.ds(s * RPT, RPT)], acc.at[pl.ds(s * RPT, RPT)])
        pltpu.sync_copy(src_hbm.at[wid], sidx)
        pltpu.sync_copy(dst_hbm.at[wid], didx)
        plsc.subcore_barrier()

        def body(j, carry):
            pltpu.async_copy(h_hbm.at[sidx.at[j]], rows, gsem).wait()
            pltpu.sync_copy(rows, acc.at[didx.at[j]], add=True)
            return carry

        nch = lax.select(c == 0, n0, n1)
        lax.fori_loop(0, nch, body, 0)
        plsc.subcore_barrier()
        pltpu.sync_copy(acc.at[pl.ds(s * RPT, RPT)],
                        out_hbm.at[c, pl.ds(s * RPT, RPT)])

    return agg(h, srcg, dstg, zeros)


def _tc_pre(x, w, b2):
    def body(x_ref, w_ref, b_ref, o_ref):
        o_ref[...] = jnp.dot(x_ref[...], w_ref[...],
                             preferred_element_type=jnp.float32) + b_ref[...]

    return pl.pallas_call(
        body, out_shape=jax.ShapeDtypeStruct((N, D), jnp.float32)
    )(x, w, b2)


def _tc_layer(h, parts, w1, w2, g2, b2):
    def body(h_ref, p_ref, w1_ref, w2_ref, g_ref, b_ref, o_ref):
        t = h_ref[...] + p_ref[0, :N] + p_ref[1, :N]
        u = jnp.dot(t, w1_ref[...], preferred_element_type=jnp.float32)
        u = jnp.where(u >= 0, u, 0.01 * u)
        z = jnp.dot(u, w2_ref[...], preferred_element_type=jnp.float32)
        m = jnp.mean(z, axis=0, keepdims=True)
        cz = z - m
        v = jnp.mean(cz * cz, axis=0, keepdims=True)
        o_ref[...] = cz * lax.rsqrt(v + 1e-4) * g_ref[...] + b_ref[...]

    return pl.pallas_call(
        body, out_shape=jax.ShapeDtypeStruct((N, D), jnp.float32)
    )(h, parts, w1, w2, g2, b2)


def _tc_final(z0, z1, z2, bt, ff1, ff2, ff3, ffsc):
    def body(z0_ref, z1_ref, z2_ref, bt_ref, f1_ref, f2_ref, f3_ref,
             fsc_ref, o_ref):
        gids = lax.broadcasted_iota(jnp.int32, (G, N), 0)
        oh = (bt_ref[...] == gids).astype(jnp.float32)
        y0 = jnp.dot(oh, z0_ref[...], preferred_element_type=jnp.float32)
        y1 = jnp.dot(oh, z1_ref[...], preferred_element_type=jnp.float32)
        y2 = jnp.dot(oh, z2_ref[...], preferred_element_type=jnp.float32)
        y = jnp.concatenate([y0, y1, y2], axis=1)

        def lk(v):
            return jnp.where(v >= 0, v, 0.01 * v)

        blk = lk(jnp.dot(y, f1_ref[...], preferred_element_type=jnp.float32))
        blk = lk(jnp.dot(blk, f2_ref[...], preferred_element_type=jnp.float32))
        blk = lk(jnp.dot(blk, f3_ref[...], preferred_element_type=jnp.float32))
        o_ref[...] = blk + jnp.dot(y, fsc_ref[...],
                                   preferred_element_type=jnp.float32)

    return pl.pallas_call(
        body, out_shape=jax.ShapeDtypeStruct((G, 3 * D), jnp.float32)
    )(z0, z1, z2, bt, ff1, ff2, ff3, ffsc)


def kernel(x, edge_index, batch, pre_W, pre_b, w1_0, w2_0, g_0, b_0,
           w1_1, w2_1, g_1, b_1, w1_2, w2_2, g_2, b_2, ff1, ff2, ff3, ffsc):
    e = edge_index.shape[1]
    totch = -(-e // (NS * CH))
    n0 = max(2, round(totch * SKEW0))
    n1 = totch - n0
    maxn = max(n0, n1)

    def slab(a, fill):
        a = a.astype(jnp.int32)
        e0 = n0 * CH * NS
        b0 = a[:e0].reshape(NS, n0, CH)
        b0 = jnp.pad(b0, ((0, 0), (0, maxn - n0), (0, 0)),
                     constant_values=fill)
        b1 = jnp.pad(a[e0:], (0, n1 * CH * NS - (e - e0)),
                     constant_values=fill).reshape(NS, n1, CH)
        b1 = jnp.pad(b1, ((0, 0), (0, maxn - n1), (0, 0)),
                     constant_values=fill)
        return jnp.concatenate([b0, b1], axis=0)

    srcg = slab(edge_index[0], 0)
    dstg = slab(edge_index[1], N)
    zeros = jnp.zeros((ACC_ROWS, D), jnp.float32)

    h = _tc_pre(x, pre_W, pre_b.reshape(1, D))
    layers = [(w1_0, w2_0, g_0, b_0), (w1_1, w2_1, g_1, b_1),
              (w1_2, w2_2, g_2, b_2)]
    zs = []
    for (w1, w2, g, b) in layers:
        parts = _sc_agg(h, srcg, dstg, zeros, n0, n1)
        h = _tc_layer(h, parts, w1, w2, g.reshape(1, D), b.reshape(1, D))
        zs.append(h)

    return _tc_final(zs[0], zs[1], zs[2],
                     batch.reshape(1, N).astype(jnp.int32),
                     ff1, ff2, ff3, ffsc)
